# Initial kernel scaffold; baseline (speedup 1.0000x reference)
import jax
import jax.numpy as jnp
from jax.experimental import pallas as pl


def kernel(features, mlp1_W, mlp1_b, lyr_W1, lyr_b1, lyr_W2, lyr_b2, gru_Wih, gru_bih, gru_Whh, gru_bhh, out0_W1, out0_b1, out0_W2, out0_b2, out1_W1, out1_b1, out1_W2, out1_b2, out2_W1, out2_b1, out2_W2, out2_b2, edge_index):
    raise NotImplementedError("write your pallas kernel here")


# speedup vs baseline: 2.5827x; 2.5827x over previous
import functools

import jax
import jax.numpy as jnp
from jax import lax
from jax.experimental import pallas as pl
from jax.experimental.pallas import tpu as pltpu
from jax.experimental.pallas import tpu_sc as plsc

N = 10000
E = 160000
NF = 256
NH = 256
HALF = 128
NLAYER = 3
NCLASSES = [2, 6, 21]

NSUB = 16
CH = 128
NCHUNK = 80
E_PAD = NSUB * NCHUNK * CH
ACC_ROWS = 10240
ZROWS = ACC_ROWS // NSUB
DUMMY_ROWS = 16

BR = 1000
GRID = N // BR


_sc_mesh = plsc.VectorSubcoreMesh(core_axis_name="c", subcore_axis_name="s")


@functools.partial(
    pl.kernel,
    out_type=(
        jax.ShapeDtypeStruct((ACC_ROWS, HALF), jnp.float32),
        jax.ShapeDtypeStruct((ACC_ROWS, HALF), jnp.float32),
    ),
    mesh=_sc_mesh,
    scratch_types=[
        pltpu.VMEM((NCHUNK, CH), jnp.int32),
        pltpu.VMEM((NCHUNK, CH), jnp.int32),
        pltpu.VMEM((CH, HALF), jnp.float32),
        pltpu.VMEM_SHARED((ACC_ROWS, HALF), jnp.float32),
        pltpu.SemaphoreType.DMA,
    ],
)
def _seg_sum(h0_hbm, h1_hbm, src_hbm, dst_hbm, zeros_hbm, out0, out1,
             src_v, dst_v, rows_v, acc, sem):
    c = lax.axis_index("c")
    s = lax.axis_index("s")

    pltpu.sync_copy(src_hbm.at[s], src_v)
    pltpu.sync_copy(dst_hbm.at[s], dst_v)
    pltpu.sync_copy(zeros_hbm, acc.at[pl.ds(s * ZROWS, ZROWS)])
    plsc.subcore_barrier()

    def run(h_ref):
        def body(j, carry):
            pltpu.async_copy(h_ref.at[src_v.at[j]], rows_v, sem).wait()
            pltpu.sync_copy(rows_v, acc.at[dst_v.at[j]], add=True)
            return carry
        lax.fori_loop(0, NCHUNK, body, 0)

    @pl.when(c == 0)
    def _():
        run(h0_hbm)

    @pl.when(c == 1)
    def _():
        run(h1_hbm)

    plsc.subcore_barrier()

    @pl.when(c == 0)
    def _():
        pltpu.sync_copy(acc.at[pl.ds(s * ZROWS, ZROWS)],
                        out0.at[pl.ds(s * ZROWS, ZROWS)])

    @pl.when(c == 1)
    def _():
        pltpu.sync_copy(acc.at[pl.ds(s * ZROWS, ZROWS)],
                        out1.at[pl.ds(s * ZROWS, ZROWS)])


def _dot(a, b):
    return jnp.dot(a, b, preferred_element_type=jnp.float32)


def _mlp1_body(x_ref, wT_ref, b_ref, o0_ref, o1_ref):
    h = jnp.maximum(_dot(x_ref[...], wT_ref[...]) + b_ref[...], 0.0)
    o0_ref[...] = h[:, :HALF]
    o1_ref[...] = h[:, HALF:]


_full = lambda shape: pl.BlockSpec(shape, lambda r: tuple(0 for _ in shape))
_rows = lambda w: pl.BlockSpec((BR, w), lambda r: (r, 0))

_mlp1_call = pl.pallas_call(
    _mlp1_body,
    grid=(GRID,),
    in_specs=[_rows(NF), _full((NF, NH)), _full((1, NH))],
    out_specs=[_rows(HALF), _rows(HALF)],
    out_shape=[jax.ShapeDtypeStruct((N, HALF), jnp.float32)] * 2,
)


def _layer_body(a0_ref, a1_ref, h0_ref, h1_ref,
                wihT_ref, bih_ref, whhT_ref, bhh_ref,
                w1T_ref, b1_ref, w2T_ref, b2_ref,
                ow1T_ref, ob1_ref, ow2T_ref, ob2_ref,
                n0_ref, n1_ref, head_ref):
    wihT = wihT_ref[...]
    whhT = whhT_ref[...]
    gi = (_dot(a0_ref[...], wihT[:HALF]) + _dot(a1_ref[...], wihT[HALF:])
          + bih_ref[...])
    gh = (_dot(h0_ref[...], whhT[:HALF]) + _dot(h1_ref[...], whhT[HALF:])
          + bhh_ref[...])
    i_r, i_z, i_n = gi[:, :NH], gi[:, NH:2 * NH], gi[:, 2 * NH:]
    h_r, h_z, h_n = gh[:, :NH], gh[:, NH:2 * NH], gh[:, 2 * NH:]
    r = jax.nn.sigmoid(i_r + h_r)
    z = jax.nn.sigmoid(i_z + h_z)
    n = jnp.tanh(i_n + r * h_n)
    hprev = jnp.concatenate([h0_ref[...], h1_ref[...]], axis=1)
    hg = (1.0 - z) * n + z * hprev
    t1 = jnp.maximum(_dot(hg, w1T_ref[...]) + b1_ref[...], 0.0)
    t2 = jnp.maximum(_dot(t1, w2T_ref[...]) + b2_ref[...], 0.0)
    n0_ref[...] = t2[:, :HALF]
    n1_ref[...] = t2[:, HALF:]
    u = jnp.maximum(_dot(t2, ow1T_ref[...]) + ob1_ref[...], 0.0)
    head_ref[...] = _dot(u, ow2T_ref[...]) + ob2_ref[...]


_layer_call = pl.pallas_call(
    _layer_body,
    grid=(GRID,),
    in_specs=[
        _rows(HALF), _rows(HALF), _rows(HALF), _rows(HALF),
        _full((NH, 3 * NH)), _full((1, 3 * NH)),
        _full((NH, 3 * NH)), _full((1, 3 * NH)),
        _full((NH, NH)), _full((1, NH)),
        _full((NH, NH)), _full((1, NH)),
        _full((NH, NH)), _full((1, NH)),
        _full((NH, HALF)), _full((1, HALF)),
    ],
    out_specs=[_rows(HALF), _rows(HALF), _rows(HALF)],
    out_shape=[
        jax.ShapeDtypeStruct((N, HALF), jnp.float32),
        jax.ShapeDtypeStruct((N, HALF), jnp.float32),
        jax.ShapeDtypeStruct((N, HALF), jnp.float32),
    ],
)


def kernel(features, mlp1_W, mlp1_b, lyr_W1, lyr_b1, lyr_W2, lyr_b2,
           gru_Wih, gru_bih, gru_Whh, gru_bhh,
           out0_W1, out0_b1, out0_W2, out0_b2,
           out1_W1, out1_b1, out1_W2, out1_b2,
           out2_W1, out2_b1, out2_W2, out2_b2,
           edge_index):
    dst = edge_index[0]
    src = edge_index[1]
    pad_n = E_PAD - E
    src_p = jnp.concatenate([src, jnp.zeros((pad_n,), jnp.int32)])
    dst_p = jnp.concatenate(
        [dst, N + (jnp.arange(pad_n, dtype=jnp.int32) % DUMMY_ROWS)])
    src3 = src_p.reshape(NSUB, NCHUNK, CH)
    dst3 = dst_p.reshape(NSUB, NCHUNK, CH)
    zeros = jnp.zeros((ZROWS, HALF), jnp.float32)

    head_params = []
    for W1, b1, W2, b2 in ((out0_W1, out0_b1, out0_W2, out0_b2),
                           (out1_W1, out1_b1, out1_W2, out1_b2),
                           (out2_W1, out2_b1, out2_W2, out2_b2)):
        nc = W2.shape[0]
        W2p = jnp.concatenate([W2, jnp.zeros((HALF - nc, NH), W2.dtype)], 0)
        b2p = jnp.concatenate([b2, jnp.zeros((HALF - nc,), b2.dtype)])
        head_params.append((W1.T, b1.reshape(1, -1), W2p.T, b2p.reshape(1, -1)))

    h0, h1 = _mlp1_call(features, mlp1_W.T, mlp1_b.reshape(1, -1))

    outputs = []
    for i in range(NLAYER):
        a0, a1 = _seg_sum(h0, h1, src3, dst3, zeros)
        ow1T, ob1, ow2T, ob2 = head_params[i]
        h0, h1, head = _layer_call(
            a0, a1, h0, h1,
            gru_Wih[i].T, gru_bih[i].reshape(1, -1),
            gru_Whh[i].T, gru_bhh[i].reshape(1, -1),
            lyr_W1[i].T, lyr_b1[i].reshape(1, -1),
            lyr_W2[i].T, lyr_b2[i].reshape(1, -1),
            ow1T, ob1, ow2T, ob2,
        )
        outputs.append(head[:, :NCLASSES[i]])
    return tuple(outputs)
